# baseline (speedup 1.0000x reference)
import jax
import jax.numpy as jnp
from jax import lax
from jax.experimental import pallas as pl
from jax.experimental.pallas import tpu as pltpu
from jax.experimental.pallas import tpu_sc as plsc

_RANK = 32
_DIM = 128
_NC = 2
_NS = 16
_NW = _NC * _NS
_CH = 128

_TBC = 3968
_TNC = 252
_TRE = 1000000 - _TNC * _TBC


def _tr_body(in_hbm, out_hbm, t_v, o_v, t2_v, o2_v, sem):
    i = pl.program_id(0)

    @pl.when(i < _TNC)
    def _full():
        cp = pltpu.make_async_copy(
            in_hbm.at[:, pl.ds(i * _TBC, _TBC)], t_v, sem)
        cp.start()
        cp.wait()
        o_v[:, :_RANK] = t_v[...].T
        cp2 = pltpu.make_async_copy(
            o_v, out_hbm.at[pl.ds(i * _TBC, _TBC), :], sem)
        cp2.start()
        cp2.wait()

    @pl.when(i == _TNC)
    def _tail():
        cp = pltpu.make_async_copy(
            in_hbm.at[:, pl.ds(_TNC * _TBC, _TRE)], t2_v, sem)
        cp.start()
        cp.wait()
        o2_v[:, :_RANK] = t2_v[...].T
        cp2 = pltpu.make_async_copy(
            o2_v, out_hbm.at[pl.ds(_TNC * _TBC, _TRE), :], sem)
        cp2.start()
        cp2.wait()


def _tc_transpose_pad(table_t):
    nrows = table_t.shape[1]
    return pl.pallas_call(
        _tr_body,
        grid=(_TNC + 1,),
        in_specs=[pl.BlockSpec(memory_space=pl.ANY)],
        out_specs=pl.BlockSpec(memory_space=pl.ANY),
        out_shape=jax.ShapeDtypeStruct((nrows, _DIM), jnp.float32),
        scratch_shapes=[
            pltpu.VMEM((_RANK, _TBC), jnp.float32),
            pltpu.VMEM((_TBC, _DIM), jnp.float32),
            pltpu.VMEM((_RANK, _TRE), jnp.float32),
            pltpu.VMEM((_TRE, _DIM), jnp.float32),
            pltpu.SemaphoreType.DMA,
        ],
    )(table_t)


_DEPTH = 4


def _gather_body(table_hbm, idx_hbm, out_hbm, idx_v, rows_v, gsem, osem):
    nchunk = idx_v.shape[0]
    wid = lax.axis_index("s") * _NC + lax.axis_index("c")
    pltpu.sync_copy(idx_hbm.at[wid], idx_v)
    gc = [None] * nchunk
    oc = [None] * nchunk
    for j in range(nchunk):
        if j >= _DEPTH:
            k = j - _DEPTH
            gc[k].wait()
            oc[k] = pltpu.async_copy(
                rows_v.at[k % _DEPTH], out_hbm.at[wid, k], osem.at[k % _DEPTH]
            )
            oc[k].wait()
        gc[j] = pltpu.async_copy(
            table_hbm.at[idx_v.at[j]], rows_v.at[j % _DEPTH],
            gsem.at[j % _DEPTH],
        )
    for k in range(nchunk - _DEPTH, nchunk):
        gc[k].wait()
        oc[k] = pltpu.async_copy(
            rows_v.at[k % _DEPTH], out_hbm.at[wid, k], osem.at[k % _DEPTH]
        )
    for k in range(nchunk - _DEPTH, nchunk):
        oc[k].wait()


def _sc_gather(table, idx3):
    nchunk = idx3.shape[1]
    fn = pl.kernel(
        _gather_body,
        out_type=jax.ShapeDtypeStruct((_NW, nchunk, _CH, _DIM), jnp.float32),
        mesh=plsc.VectorSubcoreMesh(core_axis_name="c", subcore_axis_name="s"),
        scratch_types=[
            pltpu.VMEM((nchunk, _CH), jnp.int32),
            pltpu.VMEM((_DEPTH, _CH, _DIM), jnp.float32),
            pltpu.SemaphoreType.DMA((_DEPTH,)),
            pltpu.SemaphoreType.DMA((_DEPTH,)),
        ],
        compiler_params=pltpu.CompilerParams(use_tc_tiling_on_sc=False),
    )
    return fn(table, idx3)


def _mm_body(low_ref, w_ref, b_ref, out_ref):
    out_ref[...] = (
        jnp.dot(low_ref[:, :_RANK], w_ref[...],
                preferred_element_type=jnp.float32)
        + b_ref[...]
    )


def _tc_project(low128, W, b, bm):
    n = low128.shape[0]
    return pl.pallas_call(
        _mm_body,
        grid=(n // bm,),
        in_specs=[
            pl.BlockSpec((bm, _DIM), lambda i: (i, 0)),
            pl.BlockSpec((_RANK, _DIM), lambda i: (0, 0)),
            pl.BlockSpec((1, _DIM), lambda i: (0, 0)),
        ],
        out_specs=pl.BlockSpec((bm, _DIM), lambda i: (i, 0)),
        out_shape=jax.ShapeDtypeStruct((n, _DIM), jnp.float32),
    )(low128, W, b.reshape(1, _DIM))


def kernel(x, table, W, b):
    bsz, f = x.shape
    n = bsz * f
    nchunk = n // (_NW * _CH)
    tabp = _tc_transpose_pad(table.T)
    idx3 = x.T.reshape(_NW, nchunk, _CH).astype(jnp.int32)
    low = _sc_gather(tabp, idx3)
    low128 = low.reshape(n, _DIM)
    out = _tc_project(low128, W, b, bm=2048)
    return out.reshape(f, bsz, _DIM).transpose(1, 0, 2)


# speedup vs baseline: 3.2945x; 3.2945x over previous
import jax
import jax.numpy as jnp
from jax import lax
from jax.experimental import pallas as pl
from jax.experimental.pallas import tpu as pltpu
from jax.experimental.pallas import tpu_sc as plsc

_RANK = 32
_DIM = 128
_NC = 2
_NS = 16
_NW = _NC * _NS
_CH = 128

_PBC = 15872
_PNC = 63
_PRE = 1000000 - _PNC * _PBC

_CONTRACT0 = (((0,), (0,)), ((), ()))


def _proj_body(in_hbm, w_ref, b_ref, out_hbm, t_v, o_v, t2_v, o2_v,
               isem, osem):
    i = pl.program_id(0)

    @pl.when(i == 0)
    def _prologue():
        pltpu.make_async_copy(
            in_hbm.at[:, pl.ds(0, _PBC)], t_v.at[0], isem.at[0]).start()

    @pl.when(i + 1 < _PNC)
    def _prefetch():
        nxt = i + 1
        pltpu.make_async_copy(
            in_hbm.at[:, pl.ds(nxt * _PBC, _PBC)], t_v.at[nxt % 2],
            isem.at[nxt % 2]).start()

    @pl.when(i + 1 == _PNC)
    def _prefetch_tail():
        pltpu.make_async_copy(
            in_hbm.at[:, pl.ds(_PNC * _PBC, _PRE)], t2_v,
            isem.at[_PNC % 2]).start()

    @pl.when(i < _PNC)
    def _compute():
        pltpu.make_async_copy(
            in_hbm.at[:, pl.ds(i * _PBC, _PBC)], t_v.at[i % 2],
            isem.at[i % 2]).wait()

        @pl.when(i >= 2)
        def _slot_free():
            pltpu.make_async_copy(
                o_v.at[i % 2],
                out_hbm.at[pl.ds((i - 2) * _PBC, _PBC), :],
                osem.at[i % 2]).wait()

        o_v[i % 2] = lax.dot_general(
            t_v[i % 2], w_ref[...], _CONTRACT0,
            preferred_element_type=jnp.float32,
        ) + b_ref[...]
        pltpu.make_async_copy(
            o_v.at[i % 2], out_hbm.at[pl.ds(i * _PBC, _PBC), :],
            osem.at[i % 2]).start()

    @pl.when(i == _PNC)
    def _tail():
        pltpu.make_async_copy(
            in_hbm.at[:, pl.ds(_PNC * _PBC, _PRE)], t2_v,
            isem.at[_PNC % 2]).wait()
        pltpu.make_async_copy(
            o_v.at[(_PNC - 2) % 2],
            out_hbm.at[pl.ds((_PNC - 2) * _PBC, _PBC), :],
            osem.at[(_PNC - 2) % 2]).wait()
        pltpu.make_async_copy(
            o_v.at[(_PNC - 1) % 2],
            out_hbm.at[pl.ds((_PNC - 1) * _PBC, _PBC), :],
            osem.at[(_PNC - 1) % 2]).wait()
        o2_v[...] = lax.dot_general(
            t2_v[...], w_ref[...], _CONTRACT0,
            preferred_element_type=jnp.float32,
        ) + b_ref[...]
        cp = pltpu.make_async_copy(
            o2_v, out_hbm.at[pl.ds(_PNC * _PBC, _PRE), :], osem.at[0])
        cp.start()
        cp.wait()


def _tc_project_table(table_t, W, b):
    nrows = table_t.shape[1]
    return pl.pallas_call(
        _proj_body,
        grid=(_PNC + 1,),
        in_specs=[
            pl.BlockSpec(memory_space=pl.ANY),
            pl.BlockSpec((_RANK, _DIM), lambda i: (0, 0)),
            pl.BlockSpec((1, _DIM), lambda i: (0, 0)),
        ],
        out_specs=pl.BlockSpec(memory_space=pl.ANY),
        out_shape=jax.ShapeDtypeStruct((nrows, _DIM), jnp.float32),
        scratch_shapes=[
            pltpu.VMEM((2, _RANK, _PBC), jnp.float32),
            pltpu.VMEM((2, _PBC, _DIM), jnp.float32),
            pltpu.VMEM((_RANK, _PRE), jnp.float32),
            pltpu.VMEM((_PRE, _DIM), jnp.float32),
            pltpu.SemaphoreType.DMA((2,)),
            pltpu.SemaphoreType.DMA((2,)),
        ],
    )(table_t, W, b.reshape(1, _DIM))


_DEPTH = 4


def _gather_body(table_hbm, idx_hbm, out_hbm, idx_v, rows_v, gsem, osem):
    nchunk = idx_v.shape[0]
    wid = lax.axis_index("s") * _NC + lax.axis_index("c")
    pltpu.sync_copy(idx_hbm.at[wid], idx_v)
    gc = [None] * nchunk
    oc = [None] * nchunk
    for j in range(nchunk):
        if j >= _DEPTH:
            k = j - _DEPTH
            gc[k].wait()
            oc[k] = pltpu.async_copy(
                rows_v.at[k % _DEPTH], out_hbm.at[wid, k], osem.at[k % _DEPTH]
            )
            oc[k].wait()
        gc[j] = pltpu.async_copy(
            table_hbm.at[idx_v.at[j]], rows_v.at[j % _DEPTH],
            gsem.at[j % _DEPTH],
        )
    for k in range(nchunk - _DEPTH, nchunk):
        gc[k].wait()
        oc[k] = pltpu.async_copy(
            rows_v.at[k % _DEPTH], out_hbm.at[wid, k], osem.at[k % _DEPTH]
        )
    for k in range(nchunk - _DEPTH, nchunk):
        oc[k].wait()


def _sc_gather(table, idx3):
    nchunk = idx3.shape[1]
    fn = pl.kernel(
        _gather_body,
        out_type=jax.ShapeDtypeStruct((_NW, nchunk, _CH, _DIM), jnp.float32),
        mesh=plsc.VectorSubcoreMesh(core_axis_name="c", subcore_axis_name="s"),
        scratch_types=[
            pltpu.VMEM((nchunk, _CH), jnp.int32),
            pltpu.VMEM((_DEPTH, _CH, _DIM), jnp.float32),
            pltpu.SemaphoreType.DMA((_DEPTH,)),
            pltpu.SemaphoreType.DMA((_DEPTH,)),
        ],
        compiler_params=pltpu.CompilerParams(use_tc_tiling_on_sc=False),
    )
    return fn(table, idx3)


def kernel(x, table, W, b):
    bsz, f = x.shape
    n = bsz * f
    nchunk = n // (_NW * _CH)
    proj = _tc_project_table(table.T, W, b)
    idx3 = x.T.reshape(_NW, nchunk, _CH).astype(jnp.int32)
    out = _sc_gather(proj, idx3)
    return out.reshape(f, bsz, _DIM).transpose(1, 0, 2)


# speedup vs baseline: 3.3160x; 1.0065x over previous
import jax
import jax.numpy as jnp
from jax import lax
from jax.experimental import pallas as pl
from jax.experimental.pallas import tpu as pltpu
from jax.experimental.pallas import tpu_sc as plsc

_RANK = 32
_DIM = 128
_NC = 2
_NS = 16
_NW = _NC * _NS
_CH = 128

_PBC = 15872
_PNC = 63
_PRE = 1000000 - _PNC * _PBC

_CONTRACT0 = (((0,), (0,)), ((), ()))


def _proj_body(in_hbm, w_ref, b_ref, out_hbm, t_v, o_v, t2_v, o2_v,
               isem, osem):
    i = pl.program_id(0)

    @pl.when(i == 0)
    def _prologue():
        pltpu.make_async_copy(
            in_hbm.at[:, pl.ds(0, _PBC)], t_v.at[0], isem.at[0]).start()

    @pl.when(i + 1 < _PNC)
    def _prefetch():
        nxt = i + 1
        pltpu.make_async_copy(
            in_hbm.at[:, pl.ds(nxt * _PBC, _PBC)], t_v.at[nxt % 2],
            isem.at[nxt % 2]).start()

    @pl.when(i + 1 == _PNC)
    def _prefetch_tail():
        pltpu.make_async_copy(
            in_hbm.at[:, pl.ds(_PNC * _PBC, _PRE)], t2_v,
            isem.at[_PNC % 2]).start()

    @pl.when(i < _PNC)
    def _compute():
        pltpu.make_async_copy(
            in_hbm.at[:, pl.ds(i * _PBC, _PBC)], t_v.at[i % 2],
            isem.at[i % 2]).wait()

        @pl.when(i >= 2)
        def _slot_free():
            pltpu.make_async_copy(
                o_v.at[i % 2],
                out_hbm.at[pl.ds((i - 2) * _PBC, _PBC), :],
                osem.at[i % 2]).wait()

        o_v[i % 2] = lax.dot_general(
            t_v[i % 2], w_ref[...], _CONTRACT0,
            preferred_element_type=jnp.float32,
        ) + b_ref[...]
        pltpu.make_async_copy(
            o_v.at[i % 2], out_hbm.at[pl.ds(i * _PBC, _PBC), :],
            osem.at[i % 2]).start()

    @pl.when(i == _PNC)
    def _tail():
        pltpu.make_async_copy(
            in_hbm.at[:, pl.ds(_PNC * _PBC, _PRE)], t2_v,
            isem.at[_PNC % 2]).wait()
        pltpu.make_async_copy(
            o_v.at[(_PNC - 2) % 2],
            out_hbm.at[pl.ds((_PNC - 2) * _PBC, _PBC), :],
            osem.at[(_PNC - 2) % 2]).wait()
        pltpu.make_async_copy(
            o_v.at[(_PNC - 1) % 2],
            out_hbm.at[pl.ds((_PNC - 1) * _PBC, _PBC), :],
            osem.at[(_PNC - 1) % 2]).wait()
        o2_v[...] = lax.dot_general(
            t2_v[...], w_ref[...], _CONTRACT0,
            preferred_element_type=jnp.float32,
        ) + b_ref[...]
        cp = pltpu.make_async_copy(
            o2_v, out_hbm.at[pl.ds(_PNC * _PBC, _PRE), :], osem.at[0])
        cp.start()
        cp.wait()


def _tc_project_table(table_t, W, b):
    nrows = table_t.shape[1]
    return pl.pallas_call(
        _proj_body,
        grid=(_PNC + 1,),
        in_specs=[
            pl.BlockSpec(memory_space=pl.ANY),
            pl.BlockSpec((_RANK, _DIM), lambda i: (0, 0)),
            pl.BlockSpec((1, _DIM), lambda i: (0, 0)),
        ],
        out_specs=pl.BlockSpec(memory_space=pl.ANY),
        out_shape=jax.ShapeDtypeStruct((nrows, _DIM), jnp.float32),
        scratch_shapes=[
            pltpu.VMEM((2, _RANK, _PBC), jnp.float32),
            pltpu.VMEM((2, _PBC, _DIM), jnp.float32),
            pltpu.VMEM((_RANK, _PRE), jnp.float32),
            pltpu.VMEM((_PRE, _DIM), jnp.float32),
            pltpu.SemaphoreType.DMA((2,)),
            pltpu.SemaphoreType.DMA((2,)),
        ],
    )(table_t, W, b.reshape(1, _DIM))


_DEPTH = 6


def _gather_body(table_hbm, idx_hbm, out_hbm, idx_v, rows_v, gsem, osem):
    nchunk = idx_v.shape[0]
    wid = lax.axis_index("s") * _NC + lax.axis_index("c")
    pltpu.sync_copy(idx_hbm.at[wid], idx_v)
    gc = [None] * nchunk
    oc = [None] * nchunk
    for j in range(nchunk):
        if j >= _DEPTH:
            k = j - _DEPTH
            gc[k].wait()
            oc[k] = pltpu.async_copy(
                rows_v.at[k % _DEPTH], out_hbm.at[wid, k], osem.at[k % _DEPTH]
            )
            oc[k].wait()
        gc[j] = pltpu.async_copy(
            table_hbm.at[idx_v.at[j]], rows_v.at[j % _DEPTH],
            gsem.at[j % _DEPTH],
        )
    for k in range(nchunk - _DEPTH, nchunk):
        gc[k].wait()
        oc[k] = pltpu.async_copy(
            rows_v.at[k % _DEPTH], out_hbm.at[wid, k], osem.at[k % _DEPTH]
        )
    for k in range(nchunk - _DEPTH, nchunk):
        oc[k].wait()


def _sc_gather(table, idx3):
    nchunk = idx3.shape[1]
    fn = pl.kernel(
        _gather_body,
        out_type=jax.ShapeDtypeStruct((_NW, nchunk, _CH, _DIM), jnp.float32),
        mesh=plsc.VectorSubcoreMesh(core_axis_name="c", subcore_axis_name="s"),
        scratch_types=[
            pltpu.VMEM((nchunk, _CH), jnp.int32),
            pltpu.VMEM((_DEPTH, _CH, _DIM), jnp.float32),
            pltpu.SemaphoreType.DMA((_DEPTH,)),
            pltpu.SemaphoreType.DMA((_DEPTH,)),
        ],
        compiler_params=pltpu.CompilerParams(use_tc_tiling_on_sc=False),
    )
    return fn(table, idx3)


def kernel(x, table, W, b):
    bsz, f = x.shape
    n = bsz * f
    nchunk = n // (_NW * _CH)
    proj = _tc_project_table(table.T, W, b)
    idx3 = x.T.reshape(_NW, nchunk, _CH).astype(jnp.int32)
    out = _sc_gather(proj, idx3)
    return out.reshape(f, bsz, _DIM).transpose(1, 0, 2)
